# baseline (speedup 1.0000x reference)
import functools

import jax
import jax.numpy as jnp
import numpy as np
from jax import lax
from jax.experimental import pallas as pl
from jax.experimental.pallas import tpu as pltpu
from jax.experimental.pallas import tpu_sc as plsc

NC = 2
NS = 16
L = 16
NW = NC * NS

CPAD = 1024
LOG = 10
CHUNK = CPAD // NS

_mesh = plsc.VectorSubcoreMesh(core_axis_name="c", subcore_axis_name="s")
_params = pltpu.CompilerParams(needs_layout_passes=False)


def _bitrev_np(x):
    x = ((x & 0x5555) << 1) | ((x >> 1) & 0x5555)
    x = ((x & 0x3333) << 2) | ((x >> 2) & 0x3333)
    x = ((x & 0x0F0F) << 4) | ((x >> 4) & 0x0F0F)
    x = ((x & 0x00FF) << 8) | ((x >> 8) & 0x00FF)
    return x >> (16 - LOG)


_BRTABLE = _bitrev_np(np.arange(CPAD)).astype(np.int32)


def _pad_centers(c_hbm, c_v):
    c = c_hbm.shape[0]
    pltpu.sync_copy(c_hbm, c_v.at[pl.ds(0, c)])
    inf = jnp.full((L,), jnp.inf, jnp.float32)
    for off in range(c, CPAD, L):
        c_v[pl.ds(min(off, CPAD - L), L)] = inf


def _make_fused(bpw: int, unroll: int):
    @functools.partial(
        pl.kernel,
        out_type=jax.ShapeDtypeStruct((NW, L), jnp.float32),
        mesh=_mesh,
        compiler_params=_params,
        scratch_types=[
            pltpu.VMEM((CPAD,), jnp.float32),
            pltpu.VMEM((CPAD,), jnp.int32),
            pltpu.VMEM((CPAD,), jnp.int32),
            pltpu.VMEM((CPAD,), jnp.float32),
            pltpu.VMEM((CPAD,), jnp.float32),
            pltpu.VMEM((bpw,), jnp.float32),
            pltpu.VMEM((bpw,), jnp.int32),
            pltpu.VMEM((L,), jnp.float32),
            pltpu.VMEM_SHARED((CPAD,), jnp.int32),
            pltpu.SemaphoreType.DMA,
        ],
    )
    def _fused(c_hbm, brt_hbm, x_hbm, t_hbm, out_hbm,
               c_v, brt_v, rank_v, s_v, sbr_v, x_v, t_v, acc_v,
               shr_ranks, sem):
        sid = lax.axis_index("s")
        wid = sid * NC + lax.axis_index("c")
        base = sid * CHUNK
        xbase = wid * bpw
        cps = [
            pltpu.async_copy(brt_hbm, brt_v, sem),
            pltpu.async_copy(x_hbm.at[pl.ds(xbase, bpw)], x_v, sem),
            pltpu.async_copy(t_hbm.at[pl.ds(xbase, bpw)], t_v, sem),
        ]
        _pad_centers(c_hbm, c_v)
        iota = lax.iota(jnp.int32, L)
        rots = [(iota + r) & (L - 1) for r in range(L)]
        NJ = CHUNK // L
        js = [base + v * L + iota for v in range(NJ)]
        vs = [c_v[pl.ds(base + v * L, L)] for v in range(NJ)]

        def _block(g, r, strict):
            r = list(r)
            kb = g * L
            for rot in rots:
                ck = plsc.load_gather(c_v, [kb + rot])
                for v in range(NJ):
                    cc = (ck < vs[v]) if strict else (ck <= vs[v])
                    r[v] = r[v] + cc.astype(jnp.int32)
            return tuple(r)

        zero = (jnp.zeros((L,), jnp.int32),) * NJ
        r = lax.fori_loop(0, NJ * sid, lambda g, r: _block(g, r, False), zero)
        r = list(r)
        for o in range(0, CHUNK, L):
            for rot in rots:
                kidx = base + o + rot
                ck = plsc.load_gather(c_v, [kidx])
                for v in range(NJ):
                    inc = jnp.where(kidx < js[v], ck <= vs[v], ck < vs[v])
                    r[v] = r[v] + inc.astype(jnp.int32)
        r = lax.fori_loop(NJ * (sid + 1), CPAD // L,
                          lambda g, r: _block(g, r, True), tuple(r))

        for v in range(NJ):
            rank_v[pl.ds(base + v * L, L)] = r[v]
        pltpu.sync_copy(rank_v.at[pl.ds(base, CHUNK)],
                        shr_ranks.at[pl.ds(base, CHUNK)])
        plsc.subcore_barrier()
        pltpu.sync_copy(shr_ranks, rank_v)
        for cp in cps:
            cp.wait()

        for bk in range(CPAD // L):
            rv = rank_v[pl.ds(bk * L, L)]
            cv = c_v[pl.ds(bk * L, L)]
            plsc.store_scatter(s_v, [rv], cv)
            rbr = plsc.load_gather(brt_v, [rv])
            plsc.store_scatter(sbr_v, [rbr], cv)

        def _splat(ref, a):
            return plsc.load_gather(ref, [jnp.full((L,), a, jnp.int32)])

        a1 = _splat(sbr_v, 1022)
        a2 = [_splat(sbr_v, 1020 + i) for i in range(2)]
        a3 = [_splat(sbr_v, 1016 + i) for i in range(4)]

        def one_vec(xv, tv):
            c1 = a1 <= xv
            b = c1.astype(jnp.int32)
            q = b
            p = b << (LOG - 1)
            c2 = jnp.where(c1, a2[1], a2[0]) <= xv
            b = c2.astype(jnp.int32)
            q = q | (b << 1)
            p = p | (b << (LOG - 2))
            sv3 = jnp.where(c2,
                            jnp.where(c1, a3[3], a3[2]),
                            jnp.where(c1, a3[1], a3[0]))
            b = (sv3 <= xv).astype(jnp.int32)
            q = q | (b << 2)
            p = p | (b << (LOG - 3))
            for lvl in range(4, LOG + 1):
                addr = q + (((1 << (LOG - lvl)) - 1) << lvl)
                b = (plsc.load_gather(sbr_v, [addr]) <= xv).astype(jnp.int32)
                q = q | (b << (lvl - 1))
                p = p | (b << (LOG - lvl))
            i0 = p - 2
            i1 = p - 1
            d0 = jnp.abs(xv - plsc.load_gather(s_v, [jnp.maximum(i0, 0)]))
            d1 = jnp.abs(xv - plsc.load_gather(s_v, [jnp.maximum(i1, 0)]))
            d0 = jnp.where(i0 >= 0, d0, jnp.inf)
            d1 = jnp.where(i1 >= 0, d1, jnp.inf)
            d2 = jnp.abs(xv - plsc.load_gather(s_v, [p]))
            d3 = jnp.abs(xv - plsc.load_gather(s_v, [p + 1]))
            m1 = jnp.minimum(d1, d2)
            m2 = jnp.minimum(jnp.maximum(d1, d2), jnp.where(d1 <= d2, d0, d3))
            pull = jnp.abs(xv - plsc.load_gather(c_v, [tv]))
            push = jnp.where(pull > m1, m1, m2)
            return jnp.maximum(pull - push, 0.0)

        def body(i, acc):
            for u in range(unroll):
                off = (i * unroll + u) * L
                acc = acc + one_vec(x_v[pl.ds(off, L)], t_v[pl.ds(off, L)])
            return acc

        acc = lax.fori_loop(0, bpw // (L * unroll),
                            body, jnp.zeros((L,), jnp.float32))
        acc_v[...] = acc
        pltpu.sync_copy(acc_v, out_hbm.at[wid])

    return _fused


def kernel(x, centers, transform_inds):
    b = x.shape[0]
    c = centers.shape[0]
    bpw = b // NW
    partials = _make_fused(bpw, 4)(
        centers.reshape(c), jnp.asarray(_BRTABLE),
        x.reshape(b), transform_inds)
    return jnp.sum(partials).reshape(1) / b


# speedup vs baseline: 1.0554x; 1.0554x over previous
import functools

import jax
import jax.numpy as jnp
import numpy as np
from jax import lax
from jax.experimental import pallas as pl
from jax.experimental.pallas import tpu as pltpu
from jax.experimental.pallas import tpu_sc as plsc

NC = 2
NS = 16
L = 16
NW = NC * NS

CPAD = 1024
LOG = 10
CHUNK = CPAD // NW

_mesh = plsc.VectorSubcoreMesh(core_axis_name="c", subcore_axis_name="s")
_params = pltpu.CompilerParams(needs_layout_passes=False)


def _bitrev_np(x):
    x = ((x & 0x5555) << 1) | ((x >> 1) & 0x5555)
    x = ((x & 0x3333) << 2) | ((x >> 2) & 0x3333)
    x = ((x & 0x0F0F) << 4) | ((x >> 4) & 0x0F0F)
    x = ((x & 0x00FF) << 8) | ((x >> 8) & 0x00FF)
    return x >> (16 - LOG)


_BRTABLE = _bitrev_np(np.arange(CPAD)).astype(np.int32)


def _pad_centers(c_hbm, c_v):
    c = c_hbm.shape[0]
    pltpu.sync_copy(c_hbm, c_v.at[pl.ds(0, c)])
    inf = jnp.full((L,), jnp.inf, jnp.float32)
    for off in range(c, CPAD, L):
        c_v[pl.ds(min(off, CPAD - L), L)] = inf


@functools.partial(
    pl.kernel,
    out_type=jax.ShapeDtypeStruct((CPAD,), jnp.int32),
    mesh=_mesh,
    compiler_params=_params,
    scratch_types=[
        pltpu.VMEM((CPAD,), jnp.float32),
        pltpu.VMEM((CHUNK,), jnp.int32),
    ],
)
def _sort_centers(c_hbm, rank_hbm, c_v, rank_v):
    wid = lax.axis_index("s") * NC + lax.axis_index("c")
    _pad_centers(c_hbm, c_v)
    base = wid * CHUNK
    iota = lax.iota(jnp.int32, L)
    rots = [(iota + r) & (L - 1) for r in range(L)]
    j0 = base + iota
    j1 = base + L + iota
    v0 = c_v[pl.ds(base, L)]
    v1 = c_v[pl.ds(base + L, L)]

    def _block(g, r, strict):
        r0, r1 = r
        kb = g * L
        for rot in rots:
            ck = plsc.load_gather(c_v, [kb + rot])
            c0 = (ck < v0) if strict else (ck <= v0)
            c1 = (ck < v1) if strict else (ck <= v1)
            r0 = r0 + c0.astype(jnp.int32)
            r1 = r1 + c1.astype(jnp.int32)
        return r0, r1

    zero = (jnp.zeros((L,), jnp.int32),) * 2
    r = lax.fori_loop(0, 2 * wid, lambda g, r: _block(g, r, False), zero)
    r0, r1 = r
    for o in (0, L):
        for rot in rots:
            kidx = base + o + rot
            ck = plsc.load_gather(c_v, [kidx])
            inc0 = jnp.where(kidx < j0, ck <= v0, ck < v0)
            inc1 = jnp.where(kidx < j1, ck <= v1, ck < v1)
            r0 = r0 + inc0.astype(jnp.int32)
            r1 = r1 + inc1.astype(jnp.int32)
    r = lax.fori_loop(2 * wid + 2, 2 * NW, lambda g, r: _block(g, r, True),
                      (r0, r1))
    r0, r1 = r

    rank_v[pl.ds(0, L)] = r0
    rank_v[pl.ds(L, L)] = r1
    pltpu.sync_copy(rank_v, rank_hbm.at[pl.ds(base, CHUNK)])


def _make_main(bpw: int, unroll: int):
    @functools.partial(
        pl.kernel,
        out_type=jax.ShapeDtypeStruct((NW, L), jnp.float32),
        mesh=_mesh,
        compiler_params=_params,
        scratch_types=[
            pltpu.VMEM((CPAD,), jnp.int32),
            pltpu.VMEM((CPAD,), jnp.int32),
            pltpu.VMEM((CPAD,), jnp.float32),
            pltpu.VMEM((CPAD,), jnp.float32),
            pltpu.VMEM((CPAD,), jnp.float32),
            pltpu.VMEM((bpw,), jnp.float32),
            pltpu.VMEM((bpw,), jnp.int32),
            pltpu.VMEM((L,), jnp.float32),
            pltpu.SemaphoreType.DMA,
        ],
    )
    def _main(rank_hbm, brt_hbm, c_hbm, x_hbm, t_hbm, out_hbm,
              rank_v, brt_v, c_v, s_v, sbr_v, x_v, t_v, acc_v, sem):
        wid = lax.axis_index("s") * NC + lax.axis_index("c")
        base = wid * bpw
        cps = [
            pltpu.async_copy(rank_hbm, rank_v, sem),
            pltpu.async_copy(brt_hbm, brt_v, sem),
            pltpu.async_copy(x_hbm.at[pl.ds(base, bpw)], x_v, sem),
            pltpu.async_copy(t_hbm.at[pl.ds(base, bpw)], t_v, sem),
        ]
        _pad_centers(c_hbm, c_v)
        for cp in cps:
            cp.wait()

        for b in range(CPAD // L):
            rv = rank_v[pl.ds(b * L, L)]
            cv = c_v[pl.ds(b * L, L)]
            plsc.store_scatter(s_v, [rv], cv)
            rbr = plsc.load_gather(brt_v, [rv])
            plsc.store_scatter(sbr_v, [rbr], cv)

        def _splat(ref, a):
            return plsc.load_gather(ref, [jnp.full((L,), a, jnp.int32)])

        a1 = _splat(sbr_v, 1022)
        a2 = [_splat(sbr_v, 1020 + i) for i in range(2)]
        a3 = [_splat(sbr_v, 1016 + i) for i in range(4)]

        def one_vec(xv, tv):
            c1 = a1 <= xv
            q = c1.astype(jnp.int32)
            c2 = jnp.where(c1, a2[1], a2[0]) <= xv
            q = q | (c2.astype(jnp.int32) << 1)
            sv3 = jnp.where(c2,
                            jnp.where(c1, a3[3], a3[2]),
                            jnp.where(c1, a3[1], a3[0]))
            q = q | ((sv3 <= xv).astype(jnp.int32) << 2)
            for lvl in range(4, LOG + 1):
                addr = q + (((1 << (LOG - lvl)) - 1) << lvl)
                b = (plsc.load_gather(sbr_v, [addr]) <= xv).astype(jnp.int32)
                q = q | (b << (lvl - 1))
            p = plsc.load_gather(brt_v, [q])
            i0 = p - 2
            i1 = p - 1
            d0 = jnp.abs(xv - plsc.load_gather(s_v, [jnp.maximum(i0, 0)]))
            d1 = jnp.abs(xv - plsc.load_gather(s_v, [jnp.maximum(i1, 0)]))
            d0 = jnp.where(i0 >= 0, d0, jnp.inf)
            d1 = jnp.where(i1 >= 0, d1, jnp.inf)
            d2 = jnp.abs(xv - plsc.load_gather(s_v, [p]))
            d3 = jnp.abs(xv - plsc.load_gather(s_v, [p + 1]))
            m1 = jnp.minimum(d1, d2)
            m2 = jnp.minimum(jnp.maximum(d1, d2), jnp.where(d1 <= d2, d0, d3))
            pull = jnp.abs(xv - plsc.load_gather(c_v, [tv]))
            push = jnp.where(pull > m1, m1, m2)
            return jnp.maximum(pull - push, 0.0)

        def body(i, acc):
            for u in range(unroll):
                off = (i * unroll + u) * L
                acc = acc + one_vec(x_v[pl.ds(off, L)], t_v[pl.ds(off, L)])
            return acc

        acc = lax.fori_loop(0, bpw // (L * unroll),
                            body, jnp.zeros((L,), jnp.float32))
        acc_v[...] = acc
        pltpu.sync_copy(acc_v, out_hbm.at[wid])

    return _main


def kernel(x, centers, transform_inds):
    b = x.shape[0]
    c = centers.shape[0]
    bpw = b // NW
    ranks = _sort_centers(centers.reshape(c))
    partials = _make_main(bpw, 4)(
        ranks, jnp.asarray(_BRTABLE), centers.reshape(c),
        x.reshape(b), transform_inds)
    return jnp.sum(partials).reshape(1) / b
